# baseline (speedup 1.0000x reference)
import functools

import jax
import jax.numpy as jnp
from jax import lax
from jax.experimental import pallas as pl
from jax.experimental.pallas import tpu as pltpu
from jax.experimental.pallas import tpu_sc as plsc

D = 64
K = 512
N = 64 * 32 * 32
BLK = 2048
GRID = N // BLK

NC = 2
NS = 16
NW = NC * NS
GR = 128
GPW = N // GR // NW
SLAB = 8


def _tc_assign(x_ref, w_ref, w2_ref, idx_ref, loss_ref):
    i = pl.program_id(0)
    xb = x_ref[...]
    w = w_ref[...]
    s2 = jnp.dot(xb, w2_ref[...], preferred_element_type=jnp.float32)
    xsq = jnp.sum(xb ** 2, axis=1, keepdims=True)
    wsq = jnp.sum(w ** 2, axis=0, keepdims=True)
    d = xsq - s2 + wsq
    dmin = jnp.min(d, axis=1, keepdims=True)
    iota = jnp.broadcast_to(
        lax.broadcasted_iota(jnp.int32, (1, K), 1).astype(jnp.float32), d.shape)
    idxf = jnp.min(jnp.where(d == dmin, iota, float(K)), axis=1)
    idx_ref[...] = idxf.astype(jnp.int32).reshape(idx_ref.shape)

    @pl.when(i == 0)
    def _():
        loss_ref[0, 0] = 0.0

    loss_ref[0, 0] += jnp.sum(dmin) * (2.0 / (N * D))


_tc_call = pl.pallas_call(
    _tc_assign,
    grid=(GRID,),
    in_specs=[
        pl.BlockSpec((BLK, D), lambda i: (i, 0)),
        pl.BlockSpec((D, K), lambda i: (0, 0)),
        pl.BlockSpec((D, K), lambda i: (0, 0)),
    ],
    out_specs=[
        pl.BlockSpec((8, BLK // 8), lambda i: (i, 0)),
        pl.BlockSpec(memory_space=pltpu.SMEM),
    ],
    out_shape=[
        jax.ShapeDtypeStruct((GRID * 8, BLK // 8), jnp.int32),
        jax.ShapeDtypeStruct((1, 1), jnp.float32),
    ],
)


def _sc_gather(table_hbm, idx_hbm, out_hbm, idx_v, rows_v, sem):
    wid = lax.axis_index("s") * NC + lax.axis_index("c")
    base = wid * GPW
    pltpu.sync_copy(idx_hbm.at[pl.ds(base, GPW)], idx_v)
    for h in range(GPW // SLAB):
        copies = []
        for g in range(SLAB):
            copies.append(pltpu.async_copy(
                table_hbm.at[idx_v.at[h * SLAB + g]], rows_v.at[g], sem))
        for c in copies:
            c.wait()
        pltpu.sync_copy(rows_v, out_hbm.at[pl.ds(base + h * SLAB, SLAB)])


@functools.cache
def _sc_call():
    return pl.kernel(
        _sc_gather,
        out_type=jax.ShapeDtypeStruct((N // GR, GR, D), jnp.float32),
        mesh=plsc.VectorSubcoreMesh(
            core_axis_name="c", subcore_axis_name="s",
            num_cores=NC, num_subcores=NS),
        scratch_types=[
            pltpu.VMEM((GPW, GR), jnp.int32),
            pltpu.VMEM((SLAB, GR, D), jnp.float32),
            pltpu.SemaphoreType.DMA,
        ],
        compiler_params=pltpu.CompilerParams(use_tc_tiling_on_sc=False),
    )


EB = 4
EGRID = 64 // EB


def _tc_emit(q_hbm, out_ref, qva, qvb, sema, semb):
    i = pl.program_id(0)

    def copy(j, buf, sem):
        return pltpu.make_async_copy(
            q_hbm.at[pl.ds(j * 16, 16)], buf, sem)

    @pl.when(i == 0)
    def _():
        copy(0, qva, sema).start()

    copy(2 * i + 1, qvb, semb).start()
    copy(2 * i, qva, sema).wait()
    out_ref[pl.ds(0, 2)] = qva[...].reshape(2, 32, 32, D)

    @pl.when(i + 1 < EGRID)
    def _():
        copy(2 * i + 2, qva, sema).start()

    copy(2 * i + 1, qvb, semb).wait()
    out_ref[pl.ds(2, 2)] = qvb[...].reshape(2, 32, 32, D)


_emit_call = pl.pallas_call(
    _tc_emit,
    grid=(EGRID,),
    in_specs=[pl.BlockSpec(memory_space=pltpu.MemorySpace.HBM)],
    out_specs=pl.BlockSpec((EB, 32, 32, D), lambda i: (i, 0, 0, 0)),
    out_shape=jax.ShapeDtypeStruct((64, 32, 32, D), jnp.float32),
    scratch_shapes=[
        pltpu.VMEM((16, GR, D), jnp.float32),
        pltpu.VMEM((16, GR, D), jnp.float32),
        pltpu.SemaphoreType.DMA,
        pltpu.SemaphoreType.DMA,
    ],
)


def kernel(x, w):
    flat = x.reshape(N, D)
    idx2d, loss = _tc_call(flat, w, w + w)
    wt = w.T
    idx = idx2d.reshape(N // GR, GR)
    q = _sc_call()(wt, idx)
    out = _emit_call(q)
    return out, loss[0, 0]


# speedup vs baseline: 1.3015x; 1.3015x over previous
import functools

import jax
import jax.numpy as jnp
from jax import lax
from jax.experimental import pallas as pl
from jax.experimental.pallas import tpu as pltpu
from jax.experimental.pallas import tpu_sc as plsc

D = 64
K = 512
N = 64 * 32 * 32
BLK = 2048
GRID = N // BLK

NC = 2
NS = 16
NW = NC * NS
GR = 128
GPW = N // GR // NW
SLAB = 8


def _tc_assign(x_ref, w_ref, idx_ref, loss_ref):
    i = pl.program_id(0)
    xb = x_ref[...]
    w = w_ref[...]
    s = jnp.dot(xb, w, preferred_element_type=jnp.float32)
    xsq = jnp.sum(xb ** 2, axis=1, keepdims=True)
    wsq = jnp.sum(w ** 2, axis=0, keepdims=True)
    d = xsq - 2.0 * s + wsq
    dmin = jnp.min(d, axis=1, keepdims=True)
    iota = jnp.broadcast_to(
        lax.broadcasted_iota(jnp.int32, (1, K), 1).astype(jnp.float32), d.shape)
    idxf = jnp.min(jnp.where(d == dmin, iota, float(K)), axis=1)
    idx_ref[...] = idxf.astype(jnp.int32).reshape(idx_ref.shape)

    @pl.when(i == 0)
    def _():
        loss_ref[0, 0] = 0.0

    loss_ref[0, 0] += jnp.sum(dmin) * (2.0 / (N * D))


_tc_call = pl.pallas_call(
    _tc_assign,
    grid=(GRID,),
    in_specs=[
        pl.BlockSpec((BLK, D), lambda i: (i, 0)),
        pl.BlockSpec((D, K), lambda i: (0, 0)),
    ],
    out_specs=[
        pl.BlockSpec((8, BLK // 8), lambda i: (i, 0)),
        pl.BlockSpec(memory_space=pltpu.SMEM),
    ],
    out_shape=[
        jax.ShapeDtypeStruct((GRID * 8, BLK // 8), jnp.int32),
        jax.ShapeDtypeStruct((1, 1), jnp.float32),
    ],
)


def _sc_gather(table_hbm, idx_hbm, out_hbm, idx_v, rows_v, sem):
    wid = lax.axis_index("s") * NC + lax.axis_index("c")
    base = wid * GPW
    pltpu.sync_copy(idx_hbm.at[pl.ds(base, GPW)], idx_v)
    for h in range(GPW // SLAB):
        copies = []
        for g in range(SLAB):
            copies.append(pltpu.async_copy(
                table_hbm.at[idx_v.at[h * SLAB + g]], rows_v.at[g], sem))
        for c in copies:
            c.wait()
        pltpu.sync_copy(rows_v, out_hbm.at[pl.ds(base + h * SLAB, SLAB)])


@functools.cache
def _sc_call():
    return pl.kernel(
        _sc_gather,
        out_type=jax.ShapeDtypeStruct((N // GR, GR, D), jnp.float32),
        mesh=plsc.VectorSubcoreMesh(
            core_axis_name="c", subcore_axis_name="s",
            num_cores=NC, num_subcores=NS),
        scratch_types=[
            pltpu.VMEM((GPW, GR), jnp.int32),
            pltpu.VMEM((SLAB, GR, D), jnp.float32),
            pltpu.SemaphoreType.DMA,
        ],
        compiler_params=pltpu.CompilerParams(use_tc_tiling_on_sc=False),
    )


def kernel(x, w):
    flat = x.reshape(N, D)
    idx2d, loss = _tc_call(flat, w)
    wt = w.T
    idx = idx2d.reshape(N // GR, GR)
    q = _sc_call()(wt, idx)
    out = q.reshape(x.shape)
    return out, loss[0, 0]


# speedup vs baseline: 1.3218x; 1.0156x over previous
import functools

import jax
import jax.numpy as jnp
from jax import lax
from jax.experimental import pallas as pl
from jax.experimental.pallas import tpu as pltpu
from jax.experimental.pallas import tpu_sc as plsc

D = 64
K = 512
N = 64 * 32 * 32
BLK = 4096
GRID = N // BLK

NC = 2
NS = 16
NW = NC * NS
GR = 128
GPW = N // GR // NW
SLAB = 8


def _tc_assign(x_ref, w_ref, idx_ref, loss_ref):
    i = pl.program_id(0)
    xb = x_ref[...]
    w = w_ref[...]
    s = jnp.dot(xb, w, preferred_element_type=jnp.float32)
    xsq = jnp.sum(xb ** 2, axis=1, keepdims=True)
    wsq = jnp.sum(w ** 2, axis=0, keepdims=True)
    d = xsq - 2.0 * s + wsq
    dmin = jnp.min(d, axis=1, keepdims=True)
    iota = jnp.broadcast_to(
        lax.broadcasted_iota(jnp.int32, (1, K), 1).astype(jnp.float32), d.shape)
    idxf = jnp.min(jnp.where(d == dmin, iota, float(K)), axis=1)
    idx_ref[...] = idxf.astype(jnp.int32).reshape(idx_ref.shape)

    @pl.when(i == 0)
    def _():
        loss_ref[0, 0] = 0.0

    loss_ref[0, 0] += jnp.sum(dmin) * (2.0 / (N * D))


_tc_call = pl.pallas_call(
    _tc_assign,
    grid=(GRID,),
    in_specs=[
        pl.BlockSpec((BLK, D), lambda i: (i, 0)),
        pl.BlockSpec((D, K), lambda i: (0, 0)),
    ],
    out_specs=[
        pl.BlockSpec((8, BLK // 8), lambda i: (i, 0)),
        pl.BlockSpec(memory_space=pltpu.SMEM),
    ],
    out_shape=[
        jax.ShapeDtypeStruct((GRID * 8, BLK // 8), jnp.int32),
        jax.ShapeDtypeStruct((1, 1), jnp.float32),
    ],
)


def _sc_gather(table_hbm, idx_hbm, out_hbm, idx_v, rows_v, sem):
    wid = lax.axis_index("s") * NC + lax.axis_index("c")
    base = wid * GPW
    pltpu.sync_copy(idx_hbm.at[pl.ds(base, GPW)], idx_v)
    for h in range(GPW // SLAB):
        copies = []
        for g in range(SLAB):
            copies.append(pltpu.async_copy(
                table_hbm.at[idx_v.at[h * SLAB + g]], rows_v.at[g], sem))
        for c in copies:
            c.wait()
        pltpu.sync_copy(rows_v, out_hbm.at[pl.ds(base + h * SLAB, SLAB)])


@functools.cache
def _sc_call():
    return pl.kernel(
        _sc_gather,
        out_type=jax.ShapeDtypeStruct((N // GR, GR, D), jnp.float32),
        mesh=plsc.VectorSubcoreMesh(
            core_axis_name="c", subcore_axis_name="s",
            num_cores=NC, num_subcores=NS),
        scratch_types=[
            pltpu.VMEM((GPW, GR), jnp.int32),
            pltpu.VMEM((SLAB, GR, D), jnp.float32),
            pltpu.SemaphoreType.DMA,
        ],
        compiler_params=pltpu.CompilerParams(use_tc_tiling_on_sc=False),
    )


def kernel(x, w):
    flat = x.reshape(N, D)
    idx2d, loss = _tc_call(flat, w)
    wt = w.T
    idx = idx2d.reshape(N // GR, GR)
    q = _sc_call()(wt, idx)
    out = q.reshape(x.shape)
    return out, loss[0, 0]


# speedup vs baseline: 1.3287x; 1.0052x over previous
import functools

import jax
import jax.numpy as jnp
from jax import lax
from jax.experimental import pallas as pl
from jax.experimental.pallas import tpu as pltpu
from jax.experimental.pallas import tpu_sc as plsc

D = 64
K = 512
N = 64 * 32 * 32
BLK = 8192
GRID = N // BLK

NC = 2
NS = 16
NW = NC * NS
GR = 128
GPW = N // GR // NW
SLAB = 8


def _tc_assign(x_ref, w_ref, idx_ref, loss_ref):
    i = pl.program_id(0)
    xb = x_ref[...]
    w = w_ref[...]
    s = jnp.dot(xb, w, preferred_element_type=jnp.float32)
    xsq = jnp.sum(xb ** 2, axis=1, keepdims=True)
    wsq = jnp.sum(w ** 2, axis=0, keepdims=True)
    d = xsq - 2.0 * s + wsq
    dmin = jnp.min(d, axis=1, keepdims=True)
    iota = jnp.broadcast_to(
        lax.broadcasted_iota(jnp.int32, (1, K), 1).astype(jnp.float32), d.shape)
    idxf = jnp.min(jnp.where(d == dmin, iota, float(K)), axis=1)
    idx_ref[...] = idxf.astype(jnp.int32).reshape(idx_ref.shape)

    @pl.when(i == 0)
    def _():
        loss_ref[0, 0] = 0.0

    loss_ref[0, 0] += jnp.sum(dmin) * (2.0 / (N * D))


_tc_call = pl.pallas_call(
    _tc_assign,
    grid=(GRID,),
    in_specs=[
        pl.BlockSpec((BLK, D), lambda i: (i, 0)),
        pl.BlockSpec((D, K), lambda i: (0, 0)),
    ],
    out_specs=[
        pl.BlockSpec((8, BLK // 8), lambda i: (i, 0)),
        pl.BlockSpec(memory_space=pltpu.SMEM),
    ],
    out_shape=[
        jax.ShapeDtypeStruct((GRID * 8, BLK // 8), jnp.int32),
        jax.ShapeDtypeStruct((1, 1), jnp.float32),
    ],
)


def _sc_gather(table_hbm, idx_hbm, out_hbm, idx_v, rows_v, sem):
    wid = lax.axis_index("s") * NC + lax.axis_index("c")
    base = wid * GPW
    pltpu.sync_copy(idx_hbm.at[pl.ds(base, GPW)], idx_v)
    for h in range(GPW // SLAB):
        copies = []
        for g in range(SLAB):
            copies.append(pltpu.async_copy(
                table_hbm.at[idx_v.at[h * SLAB + g]], rows_v.at[g], sem))
        for c in copies:
            c.wait()
        pltpu.sync_copy(rows_v, out_hbm.at[pl.ds(base + h * SLAB, SLAB)])


@functools.cache
def _sc_call():
    return pl.kernel(
        _sc_gather,
        out_type=jax.ShapeDtypeStruct((N // GR, GR, D), jnp.float32),
        mesh=plsc.VectorSubcoreMesh(
            core_axis_name="c", subcore_axis_name="s",
            num_cores=NC, num_subcores=NS),
        scratch_types=[
            pltpu.VMEM((GPW, GR), jnp.int32),
            pltpu.VMEM((SLAB, GR, D), jnp.float32),
            pltpu.SemaphoreType.DMA,
        ],
        compiler_params=pltpu.CompilerParams(use_tc_tiling_on_sc=False),
    )


def kernel(x, w):
    flat = x.reshape(N, D)
    idx2d, loss = _tc_call(flat, w)
    wt = w.T
    idx = idx2d.reshape(N // GR, GR)
    q = _sc_call()(wt, idx)
    out = q.reshape(x.shape)
    return out, loss[0, 0]
